# baseline (speedup 1.0000x reference)
import functools

import jax
import jax.numpy as jnp
from jax import lax
from jax.experimental import pallas as pl
from jax.experimental.pallas import tpu as pltpu
from jax.experimental.pallas import tpu_sc as plsc

N = 10000
E = 160000
DIN = 256
H = 512
DE = 16
L = 4
OUT = 128
G = 128

HC = 128
NCH = H // HC
NC = 2
NS = 16
EPC = E // NC
EPT = EPC // NS
K = 128
GB = 32
SUBB = 2
SCPTB = 20
NW = NC * NS
EPAD = NW * SCPTB * SUBB * K
NCHK = EPAD // K
BPT = EPAD // (GB * NW)
NPAD = 10112
NPW = NPAD // NS
ZROWS = 128

BN = 1000
BE = 2048


def _sc_msg_body(*refs):
    ei = refs[0]
    hn = refs[1]
    em = refs[2]
    msgs = refs[3:3 + NCH]
    (sd0, sd1, r0, r1, eb0, eb1, mo0, mo1,
     sg0, sg1, se0, se1) = refs[3 + NCH:3 + NCH + 12]
    sws = refs[3 + NCH + 12:]
    sds = (sd0, sd1)
    rows = (r0, r1)
    ebs = (eb0, eb1)
    mos = (mo0, mo1)
    sgs = (sg0, sg1)
    ses = (se0, se1)
    c = lax.axis_index("c")
    s = lax.axis_index("s")
    wid = c * NS + s

    def _issue(k, q):
        bk = wid + NW * k
        pltpu.sync_copy(ei.at[0, pl.ds(bk, 1)], sds[q])
        pltpu.async_copy(hn.at[sds[q].at[0]], rows[q], sgs[q])
        pltpu.async_copy(em.at[pl.ds(bk * GB, GB)], ebs[q], ses[q])

    def _step(k, q, prefetch, drain):
        if prefetch:
            _issue(k + 1, 1 - q)

        pltpu.make_async_copy(hn.at[sds[q].at[0]], rows[q], sgs[q]).wait()
        pltpu.make_async_copy(em.at[pl.ds(0, GB)], ebs[q], ses[q]).wait()
        if drain:
            for ci in range(NCH):
                pltpu.make_async_copy(
                    mos[q].at[pl.ds(ci * GB, GB)],
                    msgs[ci].at[pl.ds(0, GB)], sws[4 * q + ci]).wait()

        def _rw(r, cr):
            for v in range(H // 16):
                sl = pl.ds((v % (HC // 16)) * 16, 16)
                mos[q][(v // (HC // 16)) * GB + r, sl] = jnp.maximum(
                    rows[q][r, pl.ds(v * 16, 16)] + ebs[q][r, pl.ds(v * 16, 16)],
                    0.0)
            return cr
        lax.fori_loop(0, GB, _rw, 0)

        bk = wid + NW * k
        for ci in range(NCH):
            pltpu.async_copy(mos[q].at[pl.ds(ci * GB, GB)],
                             msgs[ci].at[pl.ds(bk * GB, GB)],
                             sws[4 * q + ci])

    _issue(0, 0)
    _step(0, 0, True, False)
    _step(1, 1, True, False)

    def _pair(p, carry):
        for b in range(2):
            _step(2 * p + b + 2, b, True, True)
        return carry
    lax.fori_loop(0, BPT // 2 - 2, _pair, 0)
    _step(BPT - 2, 0, True, True)
    _step(BPT - 1, 1, False, True)

    for q in range(2):
        for ci in range(NCH):
            pltpu.make_async_copy(mos[q].at[pl.ds(ci * GB, GB)],
                                  msgs[ci].at[pl.ds(0, GB)],
                                  sws[4 * q + ci]).wait()


def _sc_msg(ei32, hn, em):
    mesh = plsc.VectorSubcoreMesh(core_axis_name="c", subcore_axis_name="s",
                                  num_cores=NC, num_subcores=NS)
    fn = pl.kernel(
        _sc_msg_body,
        out_type=[jax.ShapeDtypeStruct((EPAD, HC), jnp.float32)] * NCH,
        mesh=mesh,
        scratch_types=[
            pltpu.VMEM((1, GB), jnp.int32),
            pltpu.VMEM((1, GB), jnp.int32),
            pltpu.VMEM((GB, H), jnp.float32),
            pltpu.VMEM((GB, H), jnp.float32),
            pltpu.VMEM((GB, H), jnp.float32),
            pltpu.VMEM((GB, H), jnp.float32),
            pltpu.VMEM((NCH * GB, HC), jnp.float32),
            pltpu.VMEM((NCH * GB, HC), jnp.float32),
        ] + [pltpu.SemaphoreType.DMA] * 12,
    )
    return fn(ei32, hn, em)


def _sc_agg_body(*refs):
    ei = refs[0]
    msgs = refs[1:1 + NCH]
    outs = refs[1 + NCH:1 + 2 * NCH]
    (sd, msgb, acc) = refs[1 + 2 * NCH:]
    c = lax.axis_index("c")
    s = lax.axis_index("s")
    wid = c * NS + s

    for ci in range(NCH):
        msg_c = msgs[ci]
        out_c = outs[ci]

        def _zb(i, carry):
            for j in range(HC // 16):
                msgb[i, pl.ds(j * 16, 16)] = jnp.zeros((16,), jnp.float32)
            return carry
        lax.fori_loop(0, ZROWS, _zb, 0)
        zoff = 0
        while zoff < NPW:
            zn = min(ZROWS, NPW - zoff)
            pltpu.sync_copy(msgb.at[pl.ds(0, zn)],
                            acc.at[pl.ds(s * NPW + zoff, zn)])
            zoff += zn
        plsc.subcore_barrier()

        def _super(t, carry):
            st = wid + NW * t
            ck = st * SUBB
            e0 = ck * K
            pltpu.sync_copy(ei.at[1, pl.ds(ck, SUBB)], sd)
            pltpu.sync_copy(msg_c.at[pl.ds(e0, SUBB * K)], msgb)
            for j in range(SUBB):
                pltpu.sync_copy(msgb.at[pl.ds(j * K, K)],
                                acc.at[sd.at[j]], add=True)
            return carry
        lax.fori_loop(0, SCPTB, _super, 0)

        plsc.subcore_barrier()

        pltpu.sync_copy(acc.at[pl.ds(s * NPW, NPW)],
                        out_c.at[c, pl.ds(s * NPW, NPW)])
        plsc.subcore_barrier()


def _sc_agg(ei, msgc):
    mesh = plsc.VectorSubcoreMesh(core_axis_name="c", subcore_axis_name="s",
                                  num_cores=NC, num_subcores=NS)
    fn = pl.kernel(
        _sc_agg_body,
        out_type=[jax.ShapeDtypeStruct((NC, NPAD, HC), jnp.float32)] * NCH,
        mesh=mesh,
        scratch_types=[
            pltpu.VMEM((SUBB, K), jnp.int32),
            pltpu.VMEM((SUBB * K, HC), jnp.float32),
            pltpu.VMEM_SHARED((NPAD, HC), jnp.float32),
        ],
    )
    return fn(ei, *msgc)


def _ln_block(h, scale, bias):
    m = jnp.mean(h, axis=-1, keepdims=True)
    v = jnp.mean((h - m) * (h - m), axis=-1, keepdims=True)
    return (h - m) * lax.rsqrt(v + 1e-5) * scale + bias


def _enc_body(x_ref, w_ref, b_ref, sc_ref, bi_ref,
              h_ref, hn_ref, *crs):
    h = jnp.dot(x_ref[...], w_ref[...], preferred_element_type=jnp.float32)
    h = jnp.maximum(h + b_ref[...], 0.0)
    h_ref[...] = h
    hn = _ln_block(h, sc_ref[...], bi_ref[...])
    hn_ref[...] = hn
    for i, cr in enumerate(crs):
        cr[...] = hn[:, i * HC:(i + 1) * HC]


def _encode(x, W_enc, b_enc, ln_scale, ln_bias):
    grid = (N // BN,)
    return pl.pallas_call(
        _enc_body,
        grid=grid,
        in_specs=[
            pl.BlockSpec((BN, DIN), lambda i: (i, 0)),
            pl.BlockSpec((DIN, H), lambda i: (0, 0)),
            pl.BlockSpec((1, H), lambda i: (0, 0)),
            pl.BlockSpec((1, H), lambda i: (0, 0)),
            pl.BlockSpec((1, H), lambda i: (0, 0)),
        ],
        out_specs=[
            pl.BlockSpec((BN, H), lambda i: (i, 0)),
            pl.BlockSpec((BN, H), lambda i: (i, 0)),
        ] + [pl.BlockSpec((BN, HC), lambda i: (i, 0))] * NCH,
        out_shape=[
            jax.ShapeDtypeStruct((N, H), jnp.float32),
            jax.ShapeDtypeStruct((N, H), jnp.float32),
        ] + [jax.ShapeDtypeStruct((N, HC), jnp.float32)] * NCH,
    )(x, W_enc, b_enc, ln_scale, ln_bias)


def _bond_body(ea_ref, w_ref, b_ref, o_ref):
    z = jnp.dot(ea_ref[...], w_ref[...], preferred_element_type=jnp.float32)
    o_ref[...] = z + b_ref[...]


def _bond(edge_attr, W_bond, b_bond):
    grid = (EPAD // BE,)
    return pl.pallas_call(
        _bond_body,
        grid=grid,
        in_specs=[
            pl.BlockSpec((BE, DE), lambda i: (i, 0)),
            pl.BlockSpec((DE, H), lambda i: (0, 0)),
            pl.BlockSpec((1, H), lambda i: (0, 0)),
        ],
        out_specs=pl.BlockSpec((BE, H), lambda i: (i, 0)),
        out_shape=jax.ShapeDtypeStruct((EPAD, H), jnp.float32),
    )(edge_attr, W_bond, b_bond)


def _layer_body(*args):
    h_ref, hn_ref = args[0], args[1]
    ps = args[2:2 + NCH]
    (w1_ref, b1_ref, w2_ref, b2_ref,
     eps_ref, sc_ref, bi_ref) = args[2 + NCH:9 + NCH]
    h2_ref, hn2_ref = args[9 + NCH], args[10 + NCH]
    crs = args[11 + NCH:]
    agg = jnp.concatenate(
        [p[...][0] + p[...][1] for p in ps], axis=-1)
    z = (1.0 + eps_ref[0, 0]) * hn_ref[...] + agg
    a = jnp.dot(z, w1_ref[...], preferred_element_type=jnp.float32)
    a = jnp.maximum(a + b1_ref[...], 0.0)
    zz = jnp.dot(a, w2_ref[...], preferred_element_type=jnp.float32)
    zz = zz + b2_ref[...]
    h2 = h_ref[...] + jnp.maximum(zz, 0.0)
    h2_ref[...] = h2
    hn2 = _ln_block(h2, sc_ref[...], bi_ref[...])
    hn2_ref[...] = hn2
    for i, cr in enumerate(crs):
        cr[...] = hn2[:, i * HC:(i + 1) * HC]


def _layer(h, hn, parts, W1l, b1l, W2l, b2l, epsl, ln_scale, ln_bias):
    grid = (N // BN,)
    return pl.pallas_call(
        _layer_body,
        grid=grid,
        in_specs=[
            pl.BlockSpec((BN, H), lambda i: (i, 0)),
            pl.BlockSpec((BN, H), lambda i: (i, 0)),
        ] + [pl.BlockSpec((NC, BN, HC), lambda i: (0, i, 0))] * NCH + [
            pl.BlockSpec((H, H), lambda i: (0, 0)),
            pl.BlockSpec((1, H), lambda i: (0, 0)),
            pl.BlockSpec((H, H), lambda i: (0, 0)),
            pl.BlockSpec((1, H), lambda i: (0, 0)),
            pl.BlockSpec((1, 1), lambda i: (0, 0), memory_space=pltpu.SMEM),
            pl.BlockSpec((1, H), lambda i: (0, 0)),
            pl.BlockSpec((1, H), lambda i: (0, 0)),
        ],
        out_specs=[
            pl.BlockSpec((BN, H), lambda i: (i, 0)),
            pl.BlockSpec((BN, H), lambda i: (i, 0)),
        ] + [pl.BlockSpec((BN, HC), lambda i: (i, 0))] * NCH,
        out_shape=[
            jax.ShapeDtypeStruct((N, H), jnp.float32),
            jax.ShapeDtypeStruct((N, H), jnp.float32),
        ] + [jax.ShapeDtypeStruct((N, HC), jnp.float32)] * NCH,
    )(h, hn, *parts, W1l, b1l, W2l, b2l, epsl, ln_scale, ln_bias)


def _head_body(hn_ref, b_ref, wh_ref, bh_ref, out_ref, sums, cnt):
    i = pl.program_id(0)

    @pl.when(i == 0)
    def _init():
        sums[...] = jnp.zeros_like(sums)
        cnt[...] = jnp.zeros_like(cnt)

    bvec = b_ref[0, 0, :]
    oh = (bvec[None, :] == lax.broadcasted_iota(jnp.int32, (G, BN), 0))
    oh = oh.astype(jnp.float32)
    sums[...] += jnp.dot(oh, hn_ref[...], preferred_element_type=jnp.float32)
    cnt[...] += jnp.dot(oh, jnp.ones((BN, 128), jnp.float32),
                        preferred_element_type=jnp.float32)

    @pl.when(i == (N // BN) - 1)
    def _fin():
        pooled = sums[...] / jnp.maximum(cnt[...][:, 0:1], 1.0)
        out_ref[...] = jnp.dot(pooled, wh_ref[...],
                               preferred_element_type=jnp.float32) + bh_ref[...]


def _head(hn, batch, W_head, b_head):
    nb = N // BN
    batch3 = batch.reshape(nb, 1, BN)
    return pl.pallas_call(
        _head_body,
        grid=(nb,),
        in_specs=[
            pl.BlockSpec((BN, H), lambda i: (i, 0)),
            pl.BlockSpec((1, 1, BN), lambda i: (i, 0, 0)),
            pl.BlockSpec((H, OUT), lambda i: (0, 0)),
            pl.BlockSpec((1, OUT), lambda i: (0, 0)),
        ],
        out_specs=pl.BlockSpec((G, OUT), lambda i: (0, 0)),
        out_shape=jax.ShapeDtypeStruct((G, OUT), jnp.float32),
        scratch_shapes=[
            pltpu.VMEM((G, H), jnp.float32),
            pltpu.VMEM((G, 128), jnp.float32),
        ],
    )(hn, batch3, W_head, b_head)


def kernel(x, edge_index, pestat, edge_attr, batch, W_enc, b_enc, W_bond,
           b_bond, ln_scale, ln_bias, eps, W1, b1, W2, b2, W_head, b_head):
    b_enc2 = b_enc.reshape(1, H)
    b_bond2 = b_bond.reshape(1, H)
    sc2 = ln_scale.reshape(1, H)
    bi2 = ln_bias.reshape(1, H)

    npad_e = EPAD - E
    ei_pad = jnp.concatenate(
        [edge_index,
         jnp.stack([jnp.zeros((npad_e,), jnp.int32),
                    jnp.full((npad_e,), N, jnp.int32)])], axis=1)
    ei32 = ei_pad.reshape(2, EPAD // GB, GB)
    ei_pad = ei_pad.reshape(2, NCHK, K)
    ea_pad = jnp.concatenate(
        [edge_attr, jnp.zeros((npad_e, DE), jnp.float32)], axis=0)

    embf = _bond(ea_pad, W_bond, b_bond2)
    h, hn, *hnc = _encode(x, W_enc, b_enc2, sc2, bi2)

    for l in range(L):
        msgc = _sc_msg(ei32, hn, embf)
        parts = _sc_agg(ei_pad, msgc)
        h, hn, *hnc = _layer(h, hn, parts, W1[l], b1[l].reshape(1, H),
                             W2[l], b2[l].reshape(1, H),
                             eps[l].reshape(1, 1), sc2, bi2)

    return _head(hn, batch, W_head.reshape(H, OUT), b_head.reshape(1, OUT))


# speedup vs baseline: 1.6834x; 1.6834x over previous
import functools

import jax
import jax.numpy as jnp
from jax import lax
from jax.experimental import pallas as pl
from jax.experimental.pallas import tpu as pltpu
from jax.experimental.pallas import tpu_sc as plsc

N = 10000
E = 160000
DIN = 256
H = 512
DE = 16
L = 4
OUT = 128
G = 128

HC = 128
NCH = H // HC
NC = 2
NS = 16
EPC = E // NC
EPT = EPC // NS
K = 128
GB = 32
SUBB = 2
SCPTB = 20
NW = NC * NS
EPAD = NW * SCPTB * SUBB * K
NCHK = EPAD // K
BPT = EPAD // (GB * NW)
NPAD = 10112
NPW = NPAD // NS
ZROWS = 128

BN = 1000
BE = 2048


def _sc_msg_body(*refs):
    ei = refs[0]
    hn = refs[1]
    em = refs[2]
    msgs = refs[3:3 + NCH]
    (sd0, sd1, r0, r1, eb0, eb1,
     sg0, sg1, se0, se1) = refs[3 + NCH:3 + NCH + 10]
    sws = refs[3 + NCH + 10:]
    sds = (sd0, sd1)
    rows = (r0, r1)
    ebs = (eb0, eb1)
    sgs = (sg0, sg1)
    ses = (se0, se1)
    c = lax.axis_index("c")
    s = lax.axis_index("s")
    wid = c * NS + s

    def _issue(k, q):
        bk = wid + NW * k
        pltpu.sync_copy(ei.at[0, pl.ds(bk, 1)], sds[q])
        pltpu.async_copy(hn.at[sds[q].at[0]], rows[q], sgs[q])
        pltpu.async_copy(em.at[pl.ds(bk * GB, GB)], ebs[q], ses[q])

    def _step(k, q, prefetch):
        if prefetch:
            for ci in range(NCH):
                pltpu.make_async_copy(
                    rows[1 - q].at[:, pl.ds(ci * HC, HC)],
                    msgs[ci].at[pl.ds(0, GB)], sws[4 * (1 - q) + ci]).wait()
            _issue(k + 1, 1 - q)

        pltpu.make_async_copy(hn.at[sds[q].at[0]], rows[q], sgs[q]).wait()
        pltpu.make_async_copy(em.at[pl.ds(0, GB)], ebs[q], ses[q]).wait()

        def _rw(r, cr):
            for v in range(H // 16):
                sl = pl.ds(v * 16, 16)
                rows[q][r, sl] = jnp.maximum(
                    rows[q][r, sl] + ebs[q][r, sl], 0.0)
            return cr
        lax.fori_loop(0, GB, _rw, 0)

        bk = wid + NW * k
        for ci in range(NCH):
            pltpu.async_copy(rows[q].at[:, pl.ds(ci * HC, HC)],
                             msgs[ci].at[pl.ds(bk * GB, GB)],
                             sws[4 * q + ci])

    _issue(0, 0)
    _step(0, 0, False)
    _issue(1, 1)

    def _pair(p, carry):
        for b in range(2):
            _step(2 * p + b + 1, 1 - b, True)
        return carry
    lax.fori_loop(0, BPT // 2 - 2, _pair, 0)
    _step(BPT - 3, 1, True)
    _step(BPT - 2, 0, True)
    _step(BPT - 1, 1, False)

    for q in range(2):
        for ci in range(NCH):
            pltpu.make_async_copy(rows[q].at[:, pl.ds(ci * HC, HC)],
                                  msgs[ci].at[pl.ds(0, GB)],
                                  sws[4 * q + ci]).wait()


def _sc_msg(ei32, hn, em):
    mesh = plsc.VectorSubcoreMesh(core_axis_name="c", subcore_axis_name="s",
                                  num_cores=NC, num_subcores=NS)
    fn = pl.kernel(
        _sc_msg_body,
        out_type=[jax.ShapeDtypeStruct((EPAD, HC), jnp.float32)] * NCH,
        mesh=mesh,
        scratch_types=[
            pltpu.VMEM((1, GB), jnp.int32),
            pltpu.VMEM((1, GB), jnp.int32),
            pltpu.VMEM((GB, H), jnp.float32),
            pltpu.VMEM((GB, H), jnp.float32),
            pltpu.VMEM((GB, H), jnp.float32),
            pltpu.VMEM((GB, H), jnp.float32),
        ] + [pltpu.SemaphoreType.DMA] * 12,
    )
    return fn(ei32, hn, em)


def _sc_agg_body(*refs):
    ei = refs[0]
    msgs = refs[1:1 + NCH]
    outs = refs[1 + NCH:1 + 2 * NCH]
    (sd, msgb, acc) = refs[1 + 2 * NCH:]
    c = lax.axis_index("c")
    s = lax.axis_index("s")
    wid = c * NS + s

    for ci in range(NCH):
        msg_c = msgs[ci]
        out_c = outs[ci]

        def _zb(i, carry):
            for j in range(HC // 16):
                msgb[i, pl.ds(j * 16, 16)] = jnp.zeros((16,), jnp.float32)
            return carry
        lax.fori_loop(0, ZROWS, _zb, 0)
        zoff = 0
        while zoff < NPW:
            zn = min(ZROWS, NPW - zoff)
            pltpu.sync_copy(msgb.at[pl.ds(0, zn)],
                            acc.at[pl.ds(s * NPW + zoff, zn)])
            zoff += zn
        plsc.subcore_barrier()

        def _super(t, carry):
            st = wid + NW * t
            ck = st * SUBB
            e0 = ck * K
            pltpu.sync_copy(ei.at[1, pl.ds(ck, SUBB)], sd)
            pltpu.sync_copy(msg_c.at[pl.ds(e0, SUBB * K)], msgb)
            for j in range(SUBB):
                pltpu.sync_copy(msgb.at[pl.ds(j * K, K)],
                                acc.at[sd.at[j]], add=True)
            return carry
        lax.fori_loop(0, SCPTB, _super, 0)

        plsc.subcore_barrier()

        pltpu.sync_copy(acc.at[pl.ds(s * NPW, NPW)],
                        out_c.at[c, pl.ds(s * NPW, NPW)])
        plsc.subcore_barrier()


def _sc_agg(ei, msgc):
    mesh = plsc.VectorSubcoreMesh(core_axis_name="c", subcore_axis_name="s",
                                  num_cores=NC, num_subcores=NS)
    fn = pl.kernel(
        _sc_agg_body,
        out_type=[jax.ShapeDtypeStruct((NC, NPAD, HC), jnp.float32)] * NCH,
        mesh=mesh,
        scratch_types=[
            pltpu.VMEM((SUBB, K), jnp.int32),
            pltpu.VMEM((SUBB * K, HC), jnp.float32),
            pltpu.VMEM_SHARED((NPAD, HC), jnp.float32),
        ],
    )
    return fn(ei, *msgc)


def _ln_block(h, scale, bias):
    m = jnp.mean(h, axis=-1, keepdims=True)
    v = jnp.mean((h - m) * (h - m), axis=-1, keepdims=True)
    return (h - m) * lax.rsqrt(v + 1e-5) * scale + bias


def _enc_body(x_ref, w_ref, b_ref, sc_ref, bi_ref,
              h_ref, hn_ref, *crs):
    h = jnp.dot(x_ref[...], w_ref[...], preferred_element_type=jnp.float32)
    h = jnp.maximum(h + b_ref[...], 0.0)
    h_ref[...] = h
    hn = _ln_block(h, sc_ref[...], bi_ref[...])
    hn_ref[...] = hn
    for i, cr in enumerate(crs):
        cr[...] = hn[:, i * HC:(i + 1) * HC]


def _encode(x, W_enc, b_enc, ln_scale, ln_bias):
    grid = (N // BN,)
    return pl.pallas_call(
        _enc_body,
        grid=grid,
        in_specs=[
            pl.BlockSpec((BN, DIN), lambda i: (i, 0)),
            pl.BlockSpec((DIN, H), lambda i: (0, 0)),
            pl.BlockSpec((1, H), lambda i: (0, 0)),
            pl.BlockSpec((1, H), lambda i: (0, 0)),
            pl.BlockSpec((1, H), lambda i: (0, 0)),
        ],
        out_specs=[
            pl.BlockSpec((BN, H), lambda i: (i, 0)),
            pl.BlockSpec((BN, H), lambda i: (i, 0)),
        ] + [pl.BlockSpec((BN, HC), lambda i: (i, 0))] * NCH,
        out_shape=[
            jax.ShapeDtypeStruct((N, H), jnp.float32),
            jax.ShapeDtypeStruct((N, H), jnp.float32),
        ] + [jax.ShapeDtypeStruct((N, HC), jnp.float32)] * NCH,
    )(x, W_enc, b_enc, ln_scale, ln_bias)


def _bond_body(ea_ref, w_ref, b_ref, o_ref):
    z = jnp.dot(ea_ref[...], w_ref[...], preferred_element_type=jnp.float32)
    o_ref[...] = z + b_ref[...]


def _bond(edge_attr, W_bond, b_bond):
    grid = (EPAD // BE,)
    return pl.pallas_call(
        _bond_body,
        grid=grid,
        in_specs=[
            pl.BlockSpec((BE, DE), lambda i: (i, 0)),
            pl.BlockSpec((DE, H), lambda i: (0, 0)),
            pl.BlockSpec((1, H), lambda i: (0, 0)),
        ],
        out_specs=pl.BlockSpec((BE, H), lambda i: (i, 0)),
        out_shape=jax.ShapeDtypeStruct((EPAD, H), jnp.float32),
    )(edge_attr, W_bond, b_bond)


def _layer_body(*args):
    h_ref, hn_ref = args[0], args[1]
    ps = args[2:2 + NCH]
    (w1_ref, b1_ref, w2_ref, b2_ref,
     eps_ref, sc_ref, bi_ref) = args[2 + NCH:9 + NCH]
    h2_ref, hn2_ref = args[9 + NCH], args[10 + NCH]
    crs = args[11 + NCH:]
    agg = jnp.concatenate(
        [p[...][0] + p[...][1] for p in ps], axis=-1)
    z = (1.0 + eps_ref[0, 0]) * hn_ref[...] + agg
    a = jnp.dot(z, w1_ref[...], preferred_element_type=jnp.float32)
    a = jnp.maximum(a + b1_ref[...], 0.0)
    zz = jnp.dot(a, w2_ref[...], preferred_element_type=jnp.float32)
    zz = zz + b2_ref[...]
    h2 = h_ref[...] + jnp.maximum(zz, 0.0)
    h2_ref[...] = h2
    hn2 = _ln_block(h2, sc_ref[...], bi_ref[...])
    hn2_ref[...] = hn2
    for i, cr in enumerate(crs):
        cr[...] = hn2[:, i * HC:(i + 1) * HC]


def _layer(h, hn, parts, W1l, b1l, W2l, b2l, epsl, ln_scale, ln_bias):
    grid = (N // BN,)
    return pl.pallas_call(
        _layer_body,
        grid=grid,
        in_specs=[
            pl.BlockSpec((BN, H), lambda i: (i, 0)),
            pl.BlockSpec((BN, H), lambda i: (i, 0)),
        ] + [pl.BlockSpec((NC, BN, HC), lambda i: (0, i, 0))] * NCH + [
            pl.BlockSpec((H, H), lambda i: (0, 0)),
            pl.BlockSpec((1, H), lambda i: (0, 0)),
            pl.BlockSpec((H, H), lambda i: (0, 0)),
            pl.BlockSpec((1, H), lambda i: (0, 0)),
            pl.BlockSpec((1, 1), lambda i: (0, 0), memory_space=pltpu.SMEM),
            pl.BlockSpec((1, H), lambda i: (0, 0)),
            pl.BlockSpec((1, H), lambda i: (0, 0)),
        ],
        out_specs=[
            pl.BlockSpec((BN, H), lambda i: (i, 0)),
            pl.BlockSpec((BN, H), lambda i: (i, 0)),
        ] + [pl.BlockSpec((BN, HC), lambda i: (i, 0))] * NCH,
        out_shape=[
            jax.ShapeDtypeStruct((N, H), jnp.float32),
            jax.ShapeDtypeStruct((N, H), jnp.float32),
        ] + [jax.ShapeDtypeStruct((N, HC), jnp.float32)] * NCH,
    )(h, hn, *parts, W1l, b1l, W2l, b2l, epsl, ln_scale, ln_bias)


def _head_body(hn_ref, b_ref, wh_ref, bh_ref, out_ref, sums, cnt):
    i = pl.program_id(0)

    @pl.when(i == 0)
    def _init():
        sums[...] = jnp.zeros_like(sums)
        cnt[...] = jnp.zeros_like(cnt)

    bvec = b_ref[0, 0, :]
    oh = (bvec[None, :] == lax.broadcasted_iota(jnp.int32, (G, BN), 0))
    oh = oh.astype(jnp.float32)
    sums[...] += jnp.dot(oh, hn_ref[...], preferred_element_type=jnp.float32)
    cnt[...] += jnp.dot(oh, jnp.ones((BN, 128), jnp.float32),
                        preferred_element_type=jnp.float32)

    @pl.when(i == (N // BN) - 1)
    def _fin():
        pooled = sums[...] / jnp.maximum(cnt[...][:, 0:1], 1.0)
        out_ref[...] = jnp.dot(pooled, wh_ref[...],
                               preferred_element_type=jnp.float32) + bh_ref[...]


def _head(hn, batch, W_head, b_head):
    nb = N // BN
    batch3 = batch.reshape(nb, 1, BN)
    return pl.pallas_call(
        _head_body,
        grid=(nb,),
        in_specs=[
            pl.BlockSpec((BN, H), lambda i: (i, 0)),
            pl.BlockSpec((1, 1, BN), lambda i: (i, 0, 0)),
            pl.BlockSpec((H, OUT), lambda i: (0, 0)),
            pl.BlockSpec((1, OUT), lambda i: (0, 0)),
        ],
        out_specs=pl.BlockSpec((G, OUT), lambda i: (0, 0)),
        out_shape=jax.ShapeDtypeStruct((G, OUT), jnp.float32),
        scratch_shapes=[
            pltpu.VMEM((G, H), jnp.float32),
            pltpu.VMEM((G, 128), jnp.float32),
        ],
    )(hn, batch3, W_head, b_head)


def kernel(x, edge_index, pestat, edge_attr, batch, W_enc, b_enc, W_bond,
           b_bond, ln_scale, ln_bias, eps, W1, b1, W2, b2, W_head, b_head):
    b_enc2 = b_enc.reshape(1, H)
    b_bond2 = b_bond.reshape(1, H)
    sc2 = ln_scale.reshape(1, H)
    bi2 = ln_bias.reshape(1, H)

    npad_e = EPAD - E
    ei_pad = jnp.concatenate(
        [edge_index,
         jnp.stack([jnp.zeros((npad_e,), jnp.int32),
                    jnp.full((npad_e,), N, jnp.int32)])], axis=1)
    ei32 = ei_pad.reshape(2, EPAD // GB, GB)
    ei_pad = ei_pad.reshape(2, NCHK, K)
    ea_pad = jnp.concatenate(
        [edge_attr, jnp.zeros((npad_e, DE), jnp.float32)], axis=0)

    embf = _bond(ea_pad, W_bond, b_bond2)
    h, hn, *hnc = _encode(x, W_enc, b_enc2, sc2, bi2)

    for l in range(L):
        msgc = _sc_msg(ei32, hn, embf)
        parts = _sc_agg(ei_pad, msgc)
        h, hn, *hnc = _layer(h, hn, parts, W1[l], b1[l].reshape(1, H),
                             W2[l], b2[l].reshape(1, H),
                             eps[l].reshape(1, 1), sc2, bi2)

    return _head(hn, batch, W_head.reshape(H, OUT), b_head.reshape(1, OUT))
